# Initial kernel scaffold; baseline (speedup 1.0000x reference)
import jax
import jax.numpy as jnp
from jax.experimental import pallas as pl


def kernel(input1, input2, adj_sem_ori, adj_sem_gcn, W1, b1, W3, W_ih, W_hh, b_ih, b_hh):
    raise NotImplementedError("write your pallas kernel here")


# speedup vs baseline: 30.4716x; 30.4716x over previous
import jax
import jax.numpy as jnp
from jax.experimental import pallas as pl

D = 768
BLK = 256
N_ROWS = 8192
F32 = jnp.float32


def _dot(a, b, prec=jax.lax.Precision.DEFAULT):
    return jnp.dot(a, b, preferred_element_type=F32, precision=prec)


def _dotg_hi(a, b, dims):
    return jax.lax.dot_general(a, b, dimension_numbers=(dims, ((), ())),
                               preferred_element_type=F32,
                               precision=jax.lax.Precision.HIGHEST)


def _scale_block(x1, ad):
    s_mean = jnp.mean(ad, axis=1, keepdims=True)
    s_max = jnp.max(ad, axis=1, keepdims=True)
    return jax.nn.relu(x1 * (1.0 + s_mean + s_max))


def _gru_cell(hcur, gi, gh):
    r = jax.nn.sigmoid(gi[:, 0:D] + gh[:, 0:D])
    z = jax.nn.sigmoid(gi[:, D:2 * D] + gh[:, D:2 * D])
    ng = jnp.tanh(gi[:, 2 * D:3 * D] + r * gh[:, 2 * D:3 * D])
    return (1.0 - z) * ng + z * hcur


def _head_body(x_ref, adj_ref, w1_ref, b1_ref, w3_ref, wiht_ref, whht_ref,
               bih_ref, bhh_ref, out_ref):
    xb = x_ref[...]
    hmm = _dot(xb, w1_ref[...])
    sq = jnp.sum(xb * xb, axis=1, keepdims=True)
    nrm = jnp.maximum(jnp.sqrt(sq), 1e-8)
    xn = xb / nrm
    sim = _dotg_hi(xn, xn, ((1,), (1,)))
    s_mat = (sim - jnp.min(sim)) / (jnp.max(sim) - jnp.min(sim))
    ones_c = jnp.ones((BLK, 1), F32)
    deg = _dotg_hi(s_mat, ones_c, ((0,), (0,)))
    dinv = jnp.where(deg > 0, 1.0 / jnp.sqrt(deg), 0.0)
    tmp = _dotg_hi(s_mat, dinv * hmm, ((0,), (0,)))
    x1 = jax.nn.relu(dinv * tmp + b1_ref[...])
    hcur = _scale_block(x1, adj_ref[...])
    for i in range(2):
        m = _dot(hcur, w3_ref[i])
        agg = jnp.sum(m, axis=0, keepdims=True)
        gi = _dot(agg, wiht_ref[...]) + bih_ref[...]
        gh = _dot(hcur, whht_ref[...]) + bhh_ref[...]
        hcur = _gru_cell(hcur, gi, gh)
    out_ref[...] = jax.nn.relu(hcur)


def _tail_body(x_ref, adj_ref, w1_ref, b1_ref, whht_ref, bih_ref, bhh_ref,
               out_ref):
    x1 = jax.nn.relu(_dot(x_ref[...], w1_ref[...]) + b1_ref[...])
    hcur = _scale_block(x1, adj_ref[...])
    gi = bih_ref[...]
    for _ in range(2):
        gh = _dot(hcur, whht_ref[...]) + bhh_ref[...]
        hcur = _gru_cell(hcur, gi, gh)
    out_ref[...] = jax.nn.relu(hcur)


@jax.jit
def _run(x, adjr, W1, b1r, W3, W_ihT, W_hhT, bihr, bhhr):
    head = pl.pallas_call(
        _head_body,
        grid=(1,),
        in_specs=[
            pl.BlockSpec((BLK, D), lambda i: (0, 0)),
            pl.BlockSpec((BLK, BLK), lambda i: (0, 0)),
            pl.BlockSpec((D, D), lambda i: (0, 0)),
            pl.BlockSpec((1, D), lambda i: (0, 0)),
            pl.BlockSpec((2, D, D), lambda i: (0, 0, 0)),
            pl.BlockSpec((D, 3 * D), lambda i: (0, 0)),
            pl.BlockSpec((D, 3 * D), lambda i: (0, 0)),
            pl.BlockSpec((1, 3 * D), lambda i: (0, 0)),
            pl.BlockSpec((1, 3 * D), lambda i: (0, 0)),
        ],
        out_specs=pl.BlockSpec((BLK, D), lambda i: (0, 0)),
        out_shape=jax.ShapeDtypeStruct((BLK, D), F32),
    )(x, adjr, W1, b1r, W3, W_ihT, W_hhT, bihr, bhhr)
    tail = pl.pallas_call(
        _tail_body,
        grid=(N_ROWS // BLK - 1,),
        in_specs=[
            pl.BlockSpec((BLK, D), lambda i: (i + 1, 0)),
            pl.BlockSpec((BLK, BLK), lambda i: (i + 1, 0)),
            pl.BlockSpec((D, D), lambda i: (0, 0)),
            pl.BlockSpec((1, D), lambda i: (0, 0)),
            pl.BlockSpec((D, 3 * D), lambda i: (0, 0)),
            pl.BlockSpec((1, 3 * D), lambda i: (0, 0)),
            pl.BlockSpec((1, 3 * D), lambda i: (0, 0)),
        ],
        out_specs=pl.BlockSpec((BLK, D), lambda i: (i, 0)),
        out_shape=jax.ShapeDtypeStruct((N_ROWS - BLK, D), F32),
    )(x, adjr, W1, b1r, W_hhT, bihr, bhhr)
    return jnp.concatenate([head, tail], axis=0)


def kernel(input1, input2, adj_sem_ori, adj_sem_gcn, W1, b1, W3, W_ih, W_hh,
           b_ih, b_hh):
    x = jnp.concatenate([input1.reshape(-1, D), input2.reshape(-1, D)], axis=0)
    adjr = jnp.concatenate([adj_sem_ori, adj_sem_gcn], axis=0).reshape(-1, BLK)
    out = _run(x, adjr, W1, b1.reshape(1, D), W3, W_ih.T, W_hh.T,
               b_ih.reshape(1, 3 * D), b_hh.reshape(1, 3 * D))
    xv = out.reshape(2, 16, -1, D)
    return (xv[0], xv[1])


# speedup vs baseline: 41.5737x; 1.3643x over previous
import jax
import jax.numpy as jnp
from jax.experimental import pallas as pl

D = 768
BLK = 256
F32 = jnp.float32


def _dot(a, b, prec=jax.lax.Precision.DEFAULT):
    return jnp.dot(a, b, preferred_element_type=F32, precision=prec)


def _dotg_hi(a, b, dims):
    return jax.lax.dot_general(a, b, dimension_numbers=(dims, ((), ())),
                               preferred_element_type=F32,
                               precision=jax.lax.Precision.HIGHEST)


def _scale_block(x1, ad):
    s_mean = jnp.mean(ad, axis=1, keepdims=True)
    s_max = jnp.max(ad, axis=1, keepdims=True)
    return jax.nn.relu(x1 * (1.0 + s_mean + s_max))


def _gru_cell(hcur, gi, gh):
    r = jax.nn.sigmoid(gi[:, 0:D] + gh[:, 0:D])
    z = jax.nn.sigmoid(gi[:, D:2 * D] + gh[:, D:2 * D])
    ng = jnp.tanh(gi[:, 2 * D:3 * D] + r * gh[:, 2 * D:3 * D])
    return (1.0 - z) * ng + z * hcur


def _head_body(x_ref, adj_ref, w1_ref, b1_ref, w3_ref, wiht_ref, whht_ref,
               bih_ref, bhh_ref, acc_ref, out_ref):
    del acc_ref
    xb = x_ref[...]
    hmm = _dot(xb, w1_ref[...])
    sq = jnp.sum(xb * xb, axis=1, keepdims=True)
    nrm = jnp.maximum(jnp.sqrt(sq), 1e-8)
    xn = xb / nrm
    sim = _dotg_hi(xn, xn, ((1,), (1,)))
    s_mat = (sim - jnp.min(sim)) / (jnp.max(sim) - jnp.min(sim))
    ones_c = jnp.ones((BLK, 1), F32)
    deg = _dotg_hi(s_mat, ones_c, ((0,), (0,)))
    dinv = jnp.where(deg > 0, 1.0 / jnp.sqrt(deg), 0.0)
    tmp = _dotg_hi(s_mat, dinv * hmm, ((0,), (0,)))
    x1 = jax.nn.relu(dinv * tmp + b1_ref[...])
    hcur = _scale_block(x1, adj_ref[...])
    for i in range(2):
        m = _dot(hcur, w3_ref[i])
        agg = jnp.sum(m, axis=0, keepdims=True)
        gi = _dot(agg, wiht_ref[...]) + bih_ref[...]
        gh = _dot(hcur, whht_ref[...]) + bhh_ref[...]
        hcur = _gru_cell(hcur, gi, gh)
    out_ref[...] = jax.nn.relu(hcur)


def _tail_body(x_ref, adj_ref, w1_ref, b1_ref, whht_ref, bih_ref, bhh_ref,
               out_ref):
    x1 = jax.nn.relu(_dot(x_ref[...], w1_ref[...]) + b1_ref[...])
    hcur = _scale_block(x1, adj_ref[...])
    gi = bih_ref[...]
    for _ in range(2):
        gh = _dot(hcur, whht_ref[...]) + bhh_ref[...]
        hcur = _gru_cell(hcur, gi, gh)
    out_ref[...] = jax.nn.relu(hcur)


def _tail_call(nblk, first_blk, xv, adjv, W1, b1r, W_hhT, bihr, bhhr):
    return pl.pallas_call(
        _tail_body,
        grid=(nblk,),
        in_specs=[
            pl.BlockSpec((BLK, D), lambda i: (i + first_blk, 0)),
            pl.BlockSpec((BLK, BLK), lambda i: (i + first_blk, 0)),
            pl.BlockSpec((D, D), lambda i: (0, 0)),
            pl.BlockSpec((1, D), lambda i: (0, 0)),
            pl.BlockSpec((D, 3 * D), lambda i: (0, 0)),
            pl.BlockSpec((1, 3 * D), lambda i: (0, 0)),
            pl.BlockSpec((1, 3 * D), lambda i: (0, 0)),
        ],
        out_specs=pl.BlockSpec((BLK, D), lambda i: (i + first_blk, 0)),
        out_shape=jax.ShapeDtypeStruct((xv.shape[0], D), F32),
    )(xv, adjv, W1, b1r, W_hhT, bihr, bhhr)


@jax.jit
def _run(x1v, x2v, adj1v, adj2v, W1, b1r, W3, W_ihT, W_hhT, bihr, bhhr):
    partial1 = _tail_call(15, 1, x1v, adj1v, W1, b1r, W_hhT, bihr, bhhr)
    out1 = pl.pallas_call(
        _head_body,
        grid=(1,),
        in_specs=[
            pl.BlockSpec((BLK, D), lambda i: (0, 0)),
            pl.BlockSpec((BLK, BLK), lambda i: (0, 0)),
            pl.BlockSpec((D, D), lambda i: (0, 0)),
            pl.BlockSpec((1, D), lambda i: (0, 0)),
            pl.BlockSpec((2, D, D), lambda i: (0, 0, 0)),
            pl.BlockSpec((D, 3 * D), lambda i: (0, 0)),
            pl.BlockSpec((D, 3 * D), lambda i: (0, 0)),
            pl.BlockSpec((1, 3 * D), lambda i: (0, 0)),
            pl.BlockSpec((1, 3 * D), lambda i: (0, 0)),
            pl.BlockSpec((BLK, D), lambda i: (0, 0)),
        ],
        out_specs=pl.BlockSpec((BLK, D), lambda i: (0, 0)),
        out_shape=jax.ShapeDtypeStruct((x1v.shape[0], D), F32),
        input_output_aliases={9: 0},
    )(x1v, adj1v, W1, b1r, W3, W_ihT, W_hhT, bihr, bhhr, partial1)
    out2 = _tail_call(16, 0, x2v, adj2v, W1, b1r, W_hhT, bihr, bhhr)
    return out1, out2


def kernel(input1, input2, adj_sem_ori, adj_sem_gcn, W1, b1, W3, W_ih, W_hh,
           b_ih, b_hh):
    o1, o2 = _run(input1.reshape(-1, D), input2.reshape(-1, D),
                  adj_sem_ori.reshape(-1, BLK), adj_sem_gcn.reshape(-1, BLK),
                  W1, b1.reshape(1, D), W3, W_ih.T, W_hh.T,
                  b_ih.reshape(1, 3 * D), b_hh.reshape(1, 3 * D))
    return (o1.reshape(16, BLK, D), o2.reshape(16, BLK, D))


# speedup vs baseline: 48.3976x; 1.1641x over previous
import jax
import jax.numpy as jnp
from jax.experimental import pallas as pl

D = 768
BLK = 256
F32 = jnp.float32


def _dot(a, b, prec=jax.lax.Precision.DEFAULT):
    return jnp.dot(a, b, preferred_element_type=F32, precision=prec)


def _dot_t(a, b):
    return jax.lax.dot_general(a, b, dimension_numbers=((((1,), (1,))), ((), ())),
                               preferred_element_type=F32,
                               precision=jax.lax.Precision.DEFAULT)


def _dotg_hi(a, b, dims):
    return jax.lax.dot_general(a, b, dimension_numbers=(dims, ((), ())),
                               preferred_element_type=F32,
                               precision=jax.lax.Precision.HIGHEST)


def _scale_block(x1, ad):
    s_mean = jnp.mean(ad, axis=1, keepdims=True)
    s_max = jnp.max(ad, axis=1, keepdims=True)
    return jax.nn.relu(x1 * (1.0 + s_mean + s_max))


def _gru_cell(hcur, gi, gh):
    r = jax.nn.sigmoid(gi[:, 0:D] + gh[:, 0:D])
    z = jax.nn.sigmoid(gi[:, D:2 * D] + gh[:, D:2 * D])
    ng = jnp.tanh(gi[:, 2 * D:3 * D] + r * gh[:, 2 * D:3 * D])
    return (1.0 - z) * ng + z * hcur


def _head_body(x_ref, adj_ref, w1_ref, b1_ref, w3_ref, wih_ref, whh_ref,
               bih_ref, bhh_ref, acc_ref, out_ref):
    del acc_ref
    xb = x_ref[...]
    hmm = _dot(xb, w1_ref[...])
    sq = jnp.sum(xb * xb, axis=1, keepdims=True)
    nrm = jnp.maximum(jnp.sqrt(sq), 1e-8)
    xn = xb / nrm
    sim = _dotg_hi(xn, xn, ((1,), (1,)))
    s_mat = (sim - jnp.min(sim)) / (jnp.max(sim) - jnp.min(sim))
    ones_c = jnp.ones((BLK, 1), F32)
    deg = _dotg_hi(s_mat, ones_c, ((0,), (0,)))
    dinv = jnp.where(deg > 0, 1.0 / jnp.sqrt(deg), 0.0)
    tmp = _dotg_hi(s_mat, dinv * hmm, ((0,), (0,)))
    x1 = jax.nn.relu(dinv * tmp + b1_ref[...])
    hcur = _scale_block(x1, adj_ref[...])
    for i in range(2):
        m = _dot(hcur, w3_ref[i])
        agg = jnp.sum(m, axis=0, keepdims=True)
        gi = _dot_t(agg, wih_ref[...]) + bih_ref[...]
        gh = _dot_t(hcur, whh_ref[...]) + bhh_ref[...]
        hcur = _gru_cell(hcur, gi, gh)
    out_ref[...] = jax.nn.relu(hcur)


def _tail_body(x_ref, adj_ref, w1_ref, b1_ref, whh_ref, bih_ref, bhh_ref,
               out_ref):
    x1 = jax.nn.relu(_dot(x_ref[...], w1_ref[...]) + b1_ref[...])
    hcur = _scale_block(x1, adj_ref[...])
    gi = bih_ref[...]
    for _ in range(2):
        gh = _dot_t(hcur, whh_ref[...]) + bhh_ref[...]
        hcur = _gru_cell(hcur, gi, gh)
    out_ref[...] = jax.nn.relu(hcur)


def _tail_call(nblk, first_blk, xv, adjv, W1, b1r, W_hh, bihr, bhhr):
    return pl.pallas_call(
        _tail_body,
        grid=(nblk,),
        in_specs=[
            pl.BlockSpec((BLK, D), lambda i: (i + first_blk, 0)),
            pl.BlockSpec((BLK, BLK), lambda i: (i + first_blk, 0)),
            pl.BlockSpec((D, D), lambda i: (0, 0)),
            pl.BlockSpec((1, D), lambda i: (0, 0)),
            pl.BlockSpec((3 * D, D), lambda i: (0, 0)),
            pl.BlockSpec((1, 3 * D), lambda i: (0, 0)),
            pl.BlockSpec((1, 3 * D), lambda i: (0, 0)),
        ],
        out_specs=pl.BlockSpec((BLK, D), lambda i: (i + first_blk, 0)),
        out_shape=jax.ShapeDtypeStruct((xv.shape[0], D), F32),
    )(xv, adjv, W1, b1r, W_hh, bihr, bhhr)


@jax.jit
def _run(x1v, x2v, adj1v, adj2v, W1, b1r, W3, W_ih, W_hh, bihr, bhhr):
    partial1 = _tail_call(15, 1, x1v, adj1v, W1, b1r, W_hh, bihr, bhhr)
    out1 = pl.pallas_call(
        _head_body,
        grid=(1,),
        in_specs=[
            pl.BlockSpec((BLK, D), lambda i: (0, 0)),
            pl.BlockSpec((BLK, BLK), lambda i: (0, 0)),
            pl.BlockSpec((D, D), lambda i: (0, 0)),
            pl.BlockSpec((1, D), lambda i: (0, 0)),
            pl.BlockSpec((2, D, D), lambda i: (0, 0, 0)),
            pl.BlockSpec((3 * D, D), lambda i: (0, 0)),
            pl.BlockSpec((3 * D, D), lambda i: (0, 0)),
            pl.BlockSpec((1, 3 * D), lambda i: (0, 0)),
            pl.BlockSpec((1, 3 * D), lambda i: (0, 0)),
            pl.BlockSpec((BLK, D), lambda i: (0, 0)),
        ],
        out_specs=pl.BlockSpec((BLK, D), lambda i: (0, 0)),
        out_shape=jax.ShapeDtypeStruct((x1v.shape[0], D), F32),
        input_output_aliases={9: 0},
    )(x1v, adj1v, W1, b1r, W3, W_ih, W_hh, bihr, bhhr, partial1)
    out2 = _tail_call(16, 0, x2v, adj2v, W1, b1r, W_hh, bihr, bhhr)
    return out1, out2


def kernel(input1, input2, adj_sem_ori, adj_sem_gcn, W1, b1, W3, W_ih, W_hh,
           b_ih, b_hh):
    o1, o2 = _run(input1.reshape(-1, D), input2.reshape(-1, D),
                  adj_sem_ori.reshape(-1, BLK), adj_sem_gcn.reshape(-1, BLK),
                  W1, b1.reshape(1, D), W3, W_ih, W_hh,
                  b_ih.reshape(1, 3 * D), b_hh.reshape(1, 3 * D))
    return (o1.reshape(16, BLK, D), o2.reshape(16, BLK, D))


# speedup vs baseline: 51.3923x; 1.0619x over previous
import jax
import jax.numpy as jnp
from jax.experimental import pallas as pl

D = 768
BLK = 256
F32 = jnp.float32


def _dot(a, b, prec=jax.lax.Precision.DEFAULT):
    return jnp.dot(a, b, preferred_element_type=F32, precision=prec)


def _dot_t(a, b):
    return jax.lax.dot_general(a, b, dimension_numbers=(((1,), (1,)), ((), ())),
                               preferred_element_type=F32,
                               precision=jax.lax.Precision.DEFAULT)


def _dotg_hi(a, b, dims):
    return jax.lax.dot_general(a, b, dimension_numbers=(dims, ((), ())),
                               preferred_element_type=F32,
                               precision=jax.lax.Precision.HIGHEST)


def _scale_block(x1, ad):
    s_mean = jnp.mean(ad, axis=1, keepdims=True)
    s_max = jnp.max(ad, axis=1, keepdims=True)
    return jax.nn.relu(x1 * (1.0 + s_mean + s_max))


def _gru_cell(hcur, gi, gh):
    r = jax.nn.sigmoid(gi[:, 0:D] + gh[:, 0:D])
    z = jax.nn.sigmoid(gi[:, D:2 * D] + gh[:, D:2 * D])
    ng = jnp.tanh(gi[:, 2 * D:3 * D] + r * gh[:, 2 * D:3 * D])
    return (1.0 - z) * ng + z * hcur


def _plain_block(x_ref, adj_ref, w1_ref, b1_ref, whh_ref, bih_ref, bhh_ref):
    x1 = jax.nn.relu(_dot(x_ref[...], w1_ref[...]) + b1_ref[...])
    hcur = _scale_block(x1, adj_ref[...])
    gi = bih_ref[...]
    for _ in range(2):
        gh = _dot_t(hcur, whh_ref[...]) + bhh_ref[...]
        hcur = _gru_cell(hcur, gi, gh)
    return jax.nn.relu(hcur)


def _graph_block(x_ref, adj_ref, w1_ref, b1_ref, w3_ref, wih_ref, whh_ref,
                 bih_ref, bhh_ref):
    xb = x_ref[...]
    hmm = _dot(xb, w1_ref[...])
    sq = jnp.sum(xb * xb, axis=1, keepdims=True)
    nrm = jnp.maximum(jnp.sqrt(sq), 1e-8)
    xn = xb / nrm
    sim = _dotg_hi(xn, xn, ((1,), (1,)))
    s_mat = (sim - jnp.min(sim)) / (jnp.max(sim) - jnp.min(sim))
    ones_c = jnp.ones((BLK, 1), F32)
    deg = _dotg_hi(s_mat, ones_c, ((0,), (0,)))
    dinv = jnp.where(deg > 0, 1.0 / jnp.sqrt(deg), 0.0)
    tmp = _dotg_hi(s_mat, dinv * hmm, ((0,), (0,)))
    x1 = jax.nn.relu(dinv * tmp + b1_ref[...])
    hcur = _scale_block(x1, adj_ref[...])
    for i in range(2):
        m = _dot(hcur, w3_ref[i])
        agg = jnp.sum(m, axis=0, keepdims=True)
        gi = _dot_t(agg, wih_ref[...]) + bih_ref[...]
        gh = _dot_t(hcur, whh_ref[...]) + bhh_ref[...]
        hcur = _gru_cell(hcur, gi, gh)
    return jax.nn.relu(hcur)


def _a_body(x_ref, adj_ref, w1_ref, b1_ref, w3_ref, wih_ref, whh_ref,
            bih_ref, bhh_ref, out_ref):
    pid = pl.program_id(0)

    @pl.when(pid != 15)
    def _():
        out_ref[...] = _plain_block(x_ref, adj_ref, w1_ref, b1_ref, whh_ref,
                                    bih_ref, bhh_ref)

    @pl.when(pid == 15)
    def _():
        out_ref[...] = _graph_block(x_ref, adj_ref, w1_ref, b1_ref, w3_ref,
                                    wih_ref, whh_ref, bih_ref, bhh_ref)


def _b_body(x_ref, adj_ref, w1_ref, b1_ref, whh_ref, bih_ref, bhh_ref,
            out_ref):
    out_ref[...] = _plain_block(x_ref, adj_ref, w1_ref, b1_ref, whh_ref,
                                bih_ref, bhh_ref)


@jax.jit
def _run(x1v, x2v, adj1v, adj2v, W1, b1r, W3, W_ih, W_hh, bihr, bhhr):
    out1 = pl.pallas_call(
        _a_body,
        grid=(16,),
        in_specs=[
            pl.BlockSpec((BLK, D), lambda i: ((i + 1) % 16, 0)),
            pl.BlockSpec((BLK, BLK), lambda i: ((i + 1) % 16, 0)),
            pl.BlockSpec((D, D), lambda i: (0, 0)),
            pl.BlockSpec((1, D), lambda i: (0, 0)),
            pl.BlockSpec((2, D, D), lambda i: (0, 0, 0)),
            pl.BlockSpec((3 * D, D), lambda i: (0, 0)),
            pl.BlockSpec((3 * D, D), lambda i: (0, 0)),
            pl.BlockSpec((1, 3 * D), lambda i: (0, 0)),
            pl.BlockSpec((1, 3 * D), lambda i: (0, 0)),
        ],
        out_specs=pl.BlockSpec((BLK, D), lambda i: ((i + 1) % 16, 0)),
        out_shape=jax.ShapeDtypeStruct((x1v.shape[0], D), F32),
    )(x1v, adj1v, W1, b1r, W3, W_ih, W_hh, bihr, bhhr)
    BLK2 = 512
    out2 = pl.pallas_call(
        _b_body,
        grid=(x2v.shape[0] // BLK2,),
        in_specs=[
            pl.BlockSpec((BLK2, D), lambda i: (i, 0)),
            pl.BlockSpec((BLK2, BLK), lambda i: (i, 0)),
            pl.BlockSpec((D, D), lambda i: (0, 0)),
            pl.BlockSpec((1, D), lambda i: (0, 0)),
            pl.BlockSpec((3 * D, D), lambda i: (0, 0)),
            pl.BlockSpec((1, 3 * D), lambda i: (0, 0)),
            pl.BlockSpec((1, 3 * D), lambda i: (0, 0)),
        ],
        out_specs=pl.BlockSpec((BLK2, D), lambda i: (i, 0)),
        out_shape=jax.ShapeDtypeStruct((x2v.shape[0], D), F32),
    )(x2v, adj2v, W1, b1r, W_hh, bihr, bhhr)
    return out1, out2


def kernel(input1, input2, adj_sem_ori, adj_sem_gcn, W1, b1, W3, W_ih, W_hh,
           b_ih, b_hh):
    o1, o2 = _run(input1.reshape(-1, D), input2.reshape(-1, D),
                  adj_sem_ori.reshape(-1, BLK), adj_sem_gcn.reshape(-1, BLK),
                  W1, b1.reshape(1, D), W3, W_ih, W_hh,
                  b_ih.reshape(1, 3 * D), b_hh.reshape(1, 3 * D))
    return (o1.reshape(16, BLK, D), o2.reshape(16, BLK, D))


# speedup vs baseline: 51.7078x; 1.0061x over previous
import jax
import jax.numpy as jnp
from jax.experimental import pallas as pl

D = 768
BLK = 256
F32 = jnp.float32


def _dot(a, b, prec=jax.lax.Precision.DEFAULT):
    return jnp.dot(a, b, preferred_element_type=F32, precision=prec)


def _dot_t(a, b):
    return jax.lax.dot_general(a, b, dimension_numbers=(((1,), (1,)), ((), ())),
                               preferred_element_type=F32,
                               precision=jax.lax.Precision.DEFAULT)


def _dotg_hi(a, b, dims):
    return jax.lax.dot_general(a, b, dimension_numbers=(dims, ((), ())),
                               preferred_element_type=F32,
                               precision=jax.lax.Precision.HIGHEST)


def _scale_block(x1, ad):
    s_mean = jnp.mean(ad, axis=1, keepdims=True)
    s_max = jnp.max(ad, axis=1, keepdims=True)
    return jax.nn.relu(x1 * (1.0 + s_mean + s_max))


def _gru_cell(hcur, gi, gh):
    r = jax.nn.sigmoid(gi[:, 0:D] + gh[:, 0:D])
    z = jax.nn.sigmoid(gi[:, D:2 * D] + gh[:, D:2 * D])
    ng = jnp.tanh(gi[:, 2 * D:3 * D] + r * gh[:, 2 * D:3 * D])
    return (1.0 - z) * ng + z * hcur


def _plain_block(x_ref, adj_ref, w1_ref, b1_ref, whh_ref, bih_ref, bhh_ref):
    x1 = jax.nn.relu(_dot(x_ref[...], w1_ref[...]) + b1_ref[...])
    hcur = _scale_block(x1, adj_ref[...])
    gi = bih_ref[...]
    for _ in range(2):
        gh = _dot_t(hcur, whh_ref[...]) + bhh_ref[...]
        hcur = _gru_cell(hcur, gi, gh)
    return jax.nn.relu(hcur)


def _graph_block(xb, ad, w1_ref, b1_ref, w3_ref, wih_ref, whh_ref,
                 bih_ref, bhh_ref):
    hmm = _dot(xb, w1_ref[...])
    sq = jnp.sum(xb * xb, axis=1, keepdims=True)
    nrm = jnp.maximum(jnp.sqrt(sq), 1e-8)
    xn = xb / nrm
    sim = _dotg_hi(xn, xn, ((1,), (1,)))
    s_mat = (sim - jnp.min(sim)) / (jnp.max(sim) - jnp.min(sim))
    ones_c = jnp.ones((BLK, 1), F32)
    deg = _dotg_hi(s_mat, ones_c, ((0,), (0,)))
    dinv = jnp.where(deg > 0, 1.0 / jnp.sqrt(deg), 0.0)
    tmp = _dotg_hi(s_mat, dinv * hmm, ((0,), (0,)))
    x1 = jax.nn.relu(dinv * tmp + b1_ref[...])
    hcur = _scale_block(x1, ad)
    for i in range(2):
        m = _dot(hcur, w3_ref[i])
        agg = jnp.sum(m, axis=0, keepdims=True)
        gi = _dot_t(agg, wih_ref[...]) + bih_ref[...]
        gh = _dot_t(hcur, whh_ref[...]) + bhh_ref[...]
        hcur = _gru_cell(hcur, gi, gh)
    return jax.nn.relu(hcur)


def _a_body(x_ref, adj_ref, w1_ref, b1_ref, w3_ref, wih_ref, whh_ref,
            bih_ref, bhh_ref, out_ref):
    pid = pl.program_id(0)
    out_ref[...] = _plain_block(x_ref, adj_ref, w1_ref, b1_ref, whh_ref,
                                bih_ref, bhh_ref)

    @pl.when(pid == 7)
    def _():
        out_ref[0:BLK, :] = _graph_block(
            x_ref[0:BLK, :], adj_ref[0:BLK, :], w1_ref, b1_ref, w3_ref,
            wih_ref, whh_ref, bih_ref, bhh_ref)


def _b_body(x_ref, adj_ref, w1_ref, b1_ref, whh_ref, bih_ref, bhh_ref,
            out_ref):
    out_ref[...] = _plain_block(x_ref, adj_ref, w1_ref, b1_ref, whh_ref,
                                bih_ref, bhh_ref)


@jax.jit
def _run(x1v, x2v, adj1v, adj2v, W1, b1r, W3, W_ih, W_hh, bihr, bhhr):
    BLKA = 512
    out1 = pl.pallas_call(
        _a_body,
        grid=(8,),
        in_specs=[
            pl.BlockSpec((BLKA, D), lambda i: ((i + 1) % 8, 0)),
            pl.BlockSpec((BLKA, BLK), lambda i: ((i + 1) % 8, 0)),
            pl.BlockSpec((D, D), lambda i: (0, 0)),
            pl.BlockSpec((1, D), lambda i: (0, 0)),
            pl.BlockSpec((2, D, D), lambda i: (0, 0, 0)),
            pl.BlockSpec((3 * D, D), lambda i: (0, 0)),
            pl.BlockSpec((3 * D, D), lambda i: (0, 0)),
            pl.BlockSpec((1, 3 * D), lambda i: (0, 0)),
            pl.BlockSpec((1, 3 * D), lambda i: (0, 0)),
        ],
        out_specs=pl.BlockSpec((BLKA, D), lambda i: ((i + 1) % 8, 0)),
        out_shape=jax.ShapeDtypeStruct((x1v.shape[0], D), F32),
    )(x1v, adj1v, W1, b1r, W3, W_ih, W_hh, bihr, bhhr)
    BLK2 = 512
    out2 = pl.pallas_call(
        _b_body,
        grid=(x2v.shape[0] // BLK2,),
        in_specs=[
            pl.BlockSpec((BLK2, D), lambda i: (i, 0)),
            pl.BlockSpec((BLK2, BLK), lambda i: (i, 0)),
            pl.BlockSpec((D, D), lambda i: (0, 0)),
            pl.BlockSpec((1, D), lambda i: (0, 0)),
            pl.BlockSpec((3 * D, D), lambda i: (0, 0)),
            pl.BlockSpec((1, 3 * D), lambda i: (0, 0)),
            pl.BlockSpec((1, 3 * D), lambda i: (0, 0)),
        ],
        out_specs=pl.BlockSpec((BLK2, D), lambda i: (i, 0)),
        out_shape=jax.ShapeDtypeStruct((x2v.shape[0], D), F32),
    )(x2v, adj2v, W1, b1r, W_hh, bihr, bhhr)
    return out1, out2


def kernel(input1, input2, adj_sem_ori, adj_sem_gcn, W1, b1, W3, W_ih, W_hh,
           b_ih, b_hh):
    o1, o2 = _run(input1.reshape(-1, D), input2.reshape(-1, D),
                  adj_sem_ori.reshape(-1, BLK), adj_sem_gcn.reshape(-1, BLK),
                  W1, b1.reshape(1, D), W3, W_ih, W_hh,
                  b_ih.reshape(1, 3 * D), b_hh.reshape(1, 3 * D))
    return (o1.reshape(16, BLK, D), o2.reshape(16, BLK, D))


# speedup vs baseline: 53.1887x; 1.0286x over previous
import jax
import jax.numpy as jnp
from jax.experimental import pallas as pl

D = 768
BLK = 256
F32 = jnp.float32


def _dot(a, b, prec=jax.lax.Precision.DEFAULT):
    return jnp.dot(a, b, preferred_element_type=F32, precision=prec)


def _dot_t(a, b):
    return jax.lax.dot_general(a, b, dimension_numbers=(((1,), (1,)), ((), ())),
                               preferred_element_type=F32,
                               precision=jax.lax.Precision.DEFAULT)


def _dotg_hi(a, b, dims):
    return jax.lax.dot_general(a, b, dimension_numbers=(dims, ((), ())),
                               preferred_element_type=F32,
                               precision=jax.lax.Precision.HIGHEST)


def _scale_block(x1, ad):
    s_mean = jnp.mean(ad, axis=1, keepdims=True)
    s_max = jnp.max(ad, axis=1, keepdims=True)
    return jax.nn.relu(x1 * (1.0 + s_mean + s_max))


def _gru_cell(hcur, gi, gh):
    r = jax.nn.sigmoid(gi[:, 0:D] + gh[:, 0:D])
    z = jax.nn.sigmoid(gi[:, D:2 * D] + gh[:, D:2 * D])
    ng = jnp.tanh(gi[:, 2 * D:3 * D] + r * gh[:, 2 * D:3 * D])
    return (1.0 - z) * ng + z * hcur


def _plain_block(x_ref, adj_ref, w1_ref, b1_ref, whh_ref, bih_ref, bhh_ref):
    x1 = jax.nn.relu(_dot(x_ref[...], w1_ref[...]) + b1_ref[...])
    hcur = _scale_block(x1, adj_ref[...])
    gi = bih_ref[...]
    for _ in range(2):
        gh = _dot_t(hcur, whh_ref[...]) + bhh_ref[...]
        hcur = _gru_cell(hcur, gi, gh)
    return jax.nn.relu(hcur)


def _graph_block(xb, ad, w1_ref, b1_ref, w3_ref, wih_ref, whh_ref,
                 bih_ref, bhh_ref):
    hmm = _dot(xb, w1_ref[...])
    sq = jnp.sum(xb * xb, axis=1, keepdims=True)
    nrm = jnp.maximum(jnp.sqrt(sq), 1e-8)
    xn = xb / nrm
    sim = _dotg_hi(xn, xn, ((1,), (1,)))
    s_mat = (sim - jnp.min(sim)) / (jnp.max(sim) - jnp.min(sim))
    ones_c = jnp.ones((BLK, 1), F32)
    deg = _dotg_hi(s_mat, ones_c, ((0,), (0,)))
    dinv = jnp.where(deg > 0, 1.0 / jnp.sqrt(deg), 0.0)
    tmp = _dotg_hi(s_mat, dinv * hmm, ((0,), (0,)))
    x1 = jax.nn.relu(dinv * tmp + b1_ref[...])
    hcur = _scale_block(x1, ad)
    for i in range(2):
        m = _dot(hcur, w3_ref[i])
        agg = jnp.sum(m, axis=0, keepdims=True)
        gi = _dot_t(agg, wih_ref[...]) + bih_ref[...]
        gh = _dot_t(hcur, whh_ref[...]) + bhh_ref[...]
        hcur = _gru_cell(hcur, gi, gh)
    return jax.nn.relu(hcur)


def _ab_body(x1_ref, x2_ref, adj1_ref, adj2_ref, w1_ref, b1_ref, w3_ref,
             wih_ref, whh_ref, bih_ref, bhh_ref, o1_ref, o2_ref):
    pid = pl.program_id(0)
    is1 = pid < 8
    xb = jnp.where(is1, x1_ref[...], x2_ref[...])
    ad = jnp.where(is1, adj1_ref[...], adj2_ref[...])
    res = _plain_block(xb, ad, w1_ref, b1_ref, whh_ref, bih_ref, bhh_ref)

    @pl.when(is1)
    def _():
        o1_ref[...] = res

    @pl.when(pid == 7)
    def _():
        o1_ref[0:BLK, :] = _graph_block(
            x1_ref[0:BLK, :], adj1_ref[0:BLK, :], w1_ref, b1_ref, w3_ref,
            wih_ref, whh_ref, bih_ref, bhh_ref)

    @pl.when(jnp.logical_not(is1))
    def _():
        o2_ref[...] = res


@jax.jit
def _run(x1v, x2v, adj1v, adj2v, W1, b1r, W3, W_ih, W_hh, bihr, bhhr):
    BLKA = 512
    i1_map = lambda i: (jnp.where(i < 8, (i + 1) % 8, 0), 0)
    i2_map = lambda i: (jnp.where(i < 8, 0, i - 8), 0)
    out1, out2 = pl.pallas_call(
        _ab_body,
        grid=(16,),
        in_specs=[
            pl.BlockSpec((BLKA, D), i1_map),
            pl.BlockSpec((BLKA, D), i2_map),
            pl.BlockSpec((BLKA, BLK), i1_map),
            pl.BlockSpec((BLKA, BLK), i2_map),
            pl.BlockSpec((D, D), lambda i: (0, 0)),
            pl.BlockSpec((1, D), lambda i: (0, 0)),
            pl.BlockSpec((2, D, D), lambda i: (0, 0, 0)),
            pl.BlockSpec((3 * D, D), lambda i: (0, 0)),
            pl.BlockSpec((3 * D, D), lambda i: (0, 0)),
            pl.BlockSpec((1, 3 * D), lambda i: (0, 0)),
            pl.BlockSpec((1, 3 * D), lambda i: (0, 0)),
        ],
        out_specs=[
            pl.BlockSpec((BLKA, D), i1_map),
            pl.BlockSpec((BLKA, D), i2_map),
        ],
        out_shape=[
            jax.ShapeDtypeStruct((x1v.shape[0], D), F32),
            jax.ShapeDtypeStruct((x2v.shape[0], D), F32),
        ],
    )(x1v, x2v, adj1v, adj2v, W1, b1r, W3, W_ih, W_hh, bihr, bhhr)
    return out1, out2


def kernel(input1, input2, adj_sem_ori, adj_sem_gcn, W1, b1, W3, W_ih, W_hh,
           b_ih, b_hh):
    o1, o2 = _run(input1.reshape(-1, D), input2.reshape(-1, D),
                  adj_sem_ori.reshape(-1, BLK), adj_sem_gcn.reshape(-1, BLK),
                  W1, b1.reshape(1, D), W3, W_ih, W_hh,
                  b_ih.reshape(1, 3 * D), b_hh.reshape(1, 3 * D))
    return (o1.reshape(16, BLK, D), o2.reshape(16, BLK, D))


# speedup vs baseline: 55.4165x; 1.0419x over previous
import jax
import jax.numpy as jnp
from jax.experimental import pallas as pl
from jax.experimental.pallas import tpu as pltpu

D = 768
BLK = 256
F32 = jnp.float32


def _dot(a, b, prec=jax.lax.Precision.DEFAULT):
    return jnp.dot(a, b, preferred_element_type=F32, precision=prec)


def _dot_t(a, b):
    return jax.lax.dot_general(a, b, dimension_numbers=(((1,), (1,)), ((), ())),
                               preferred_element_type=F32,
                               precision=jax.lax.Precision.DEFAULT)


def _dotg_hi(a, b, dims):
    return jax.lax.dot_general(a, b, dimension_numbers=(dims, ((), ())),
                               preferred_element_type=F32,
                               precision=jax.lax.Precision.HIGHEST)


def _scale_block(x1, ad):
    s_mean = jnp.mean(ad, axis=1, keepdims=True)
    s_max = jnp.max(ad, axis=1, keepdims=True)
    return jax.nn.relu(x1 * (1.0 + s_mean + s_max))


def _gru_cell(hcur, gi, gh):
    r = jax.nn.sigmoid(gi[:, 0:D] + gh[:, 0:D])
    z = jax.nn.sigmoid(gi[:, D:2 * D] + gh[:, D:2 * D])
    ng = jnp.tanh(gi[:, 2 * D:3 * D] + r * gh[:, 2 * D:3 * D])
    return (1.0 - z) * ng + z * hcur


def _plain_block(x_ref, adj_ref, w1_ref, b1_ref, whh_ref, bih_ref, bhh_ref):
    x1 = jax.nn.relu(_dot(x_ref[...], w1_ref[...]) + b1_ref[...])
    hcur = _scale_block(x1, adj_ref[...])
    gi = bih_ref[...]
    for _ in range(2):
        gh = _dot_t(hcur, whh_ref[...]) + bhh_ref[...]
        hcur = _gru_cell(hcur, gi, gh)
    return jax.nn.relu(hcur)


def _graph_block(xb, ad, w1_ref, b1_ref, w3_ref, wih_ref, whh_ref,
                 bih_ref, bhh_ref):
    hmm = _dot(xb, w1_ref[...])
    sq = jnp.sum(xb * xb, axis=1, keepdims=True)
    nrm = jnp.maximum(jnp.sqrt(sq), 1e-8)
    xn = xb / nrm
    sim = _dotg_hi(xn, xn, ((1,), (1,)))
    s_mat = (sim - jnp.min(sim)) / (jnp.max(sim) - jnp.min(sim))
    ones_c = jnp.ones((BLK, 1), F32)
    deg = _dotg_hi(s_mat, ones_c, ((0,), (0,)))
    dinv = jnp.where(deg > 0, 1.0 / jnp.sqrt(deg), 0.0)
    tmp = _dotg_hi(s_mat, dinv * hmm, ((0,), (0,)))
    x1 = jax.nn.relu(dinv * tmp + b1_ref[...])
    hcur = _scale_block(x1, ad)
    for i in range(2):
        m = _dot(hcur, w3_ref[i])
        agg = jnp.sum(m, axis=0, keepdims=True)
        gi = _dot_t(agg, wih_ref[...]) + bih_ref[...]
        gh = _dot_t(hcur, whh_ref[...]) + bhh_ref[...]
        hcur = _gru_cell(hcur, gi, gh)
    return jax.nn.relu(hcur)


def _ab_body(x1_ref, x2_ref, adj1_ref, adj2_ref, w1_ref, b1_ref, w3_hbm,
             wih_hbm, whh_ref, bih_ref, bhh_ref, o1_ref, o2_ref,
             w3_vm, wih_vm, sem3, semih):
    pid = pl.program_id(0)

    @pl.when(pid == 0)
    def _():
        pltpu.make_async_copy(w3_hbm, w3_vm, sem3).start()
        pltpu.make_async_copy(wih_hbm, wih_vm, semih).start()

    @pl.when(pid < 8)
    def _():
        o1_ref[...] = _plain_block(x1_ref, adj1_ref, w1_ref, b1_ref,
                                   whh_ref, bih_ref, bhh_ref)

    @pl.when(pid == 7)
    def _():
        pltpu.make_async_copy(w3_hbm, w3_vm, sem3).wait()
        pltpu.make_async_copy(wih_hbm, wih_vm, semih).wait()
        o1_ref[0:BLK, :] = _graph_block(
            x1_ref[0:BLK, :], adj1_ref[0:BLK, :], w1_ref, b1_ref, w3_vm,
            wih_vm, whh_ref, bih_ref, bhh_ref)

    @pl.when(pid >= 8)
    def _():
        o2_ref[...] = _plain_block(x2_ref, adj2_ref, w1_ref, b1_ref,
                                   whh_ref, bih_ref, bhh_ref)


@jax.jit
def _run(x1v, x2v, adj1v, adj2v, W1, b1r, W3, W_ih, W_hh, bihr, bhhr):
    BLKA = 512
    i1_map = lambda i: (jnp.where(i < 8, (i + 1) % 8, 0), 0)
    i2_map = lambda i: (jnp.where(i < 8, 0, i - 8), 0)
    out1, out2 = pl.pallas_call(
        _ab_body,
        grid=(16,),
        in_specs=[
            pl.BlockSpec((BLKA, D), i1_map),
            pl.BlockSpec((BLKA, D), i2_map),
            pl.BlockSpec((BLKA, BLK), i1_map),
            pl.BlockSpec((BLKA, BLK), i2_map),
            pl.BlockSpec((D, D), lambda i: (0, 0)),
            pl.BlockSpec((1, D), lambda i: (0, 0)),
            pl.BlockSpec(memory_space=pltpu.MemorySpace.HBM),
            pl.BlockSpec(memory_space=pltpu.MemorySpace.HBM),
            pl.BlockSpec((3 * D, D), lambda i: (0, 0)),
            pl.BlockSpec((1, 3 * D), lambda i: (0, 0)),
            pl.BlockSpec((1, 3 * D), lambda i: (0, 0)),
        ],
        out_specs=[
            pl.BlockSpec((BLKA, D), i1_map),
            pl.BlockSpec((BLKA, D), i2_map),
        ],
        out_shape=[
            jax.ShapeDtypeStruct((x1v.shape[0], D), F32),
            jax.ShapeDtypeStruct((x2v.shape[0], D), F32),
        ],
        scratch_shapes=[
            pltpu.VMEM((2, D, D), F32),
            pltpu.VMEM((3 * D, D), F32),
            pltpu.SemaphoreType.DMA,
            pltpu.SemaphoreType.DMA,
        ],
    )(x1v, x2v, adj1v, adj2v, W1, b1r, W3, W_ih, W_hh, bihr, bhhr)
    return out1, out2


def kernel(input1, input2, adj_sem_ori, adj_sem_gcn, W1, b1, W3, W_ih, W_hh,
           b_ih, b_hh):
    o1, o2 = _run(input1.reshape(-1, D), input2.reshape(-1, D),
                  adj_sem_ori.reshape(-1, BLK), adj_sem_gcn.reshape(-1, BLK),
                  W1, b1.reshape(1, D), W3, W_ih, W_hh,
                  b_ih.reshape(1, 3 * D), b_hh.reshape(1, 3 * D))
    return (o1.reshape(16, BLK, D), o2.reshape(16, BLK, D))
